# Initial kernel scaffold; baseline (speedup 1.0000x reference)
import jax
import jax.numpy as jnp
from jax.experimental import pallas as pl


def kernel(x, edge_index, W0, U0, c0, b0, gamma0, beta0, W1, U1, c1, b1):
    raise NotImplementedError("write your pallas kernel here")


# speedup vs baseline: 3.6950x; 3.6950x over previous
import functools

import jax
import jax.numpy as jnp
from jax import lax
from jax.experimental import pallas as pl
from jax.experimental.pallas import tpu as pltpu
from jax.experimental.pallas import tpu_sc as plsc

N = 10000
E = 320000
D = 128
H = 4
C = 128
HC = H * C
XUW = 128
GW = HC + XUW
KCAT = GW + C

NC, NS = 2, 16
NW = NC * NS
EB = 32
NCHUNK = 313
EPW = EB * NCHUNK
EPAD = NW * EPW
NA = N + 8
RA = 632
RB = N - (NS - 1) * RA

_F32 = jnp.float32


def _mm_body(x_ref, w_ref, comb_ref, xud_ref, self_ref):
    y = jnp.dot(x_ref[...], w_ref[...], preferred_element_type=_F32)
    comb_ref[...] = y[:, :GW]
    xud_ref[...] = y[:, HC:GW]
    self_ref[...] = y[:, GW:]


def _mm_affine_body(x_ref, a_ref, b_ref, w_ref, comb_ref, xud_ref, self_ref):
    xb = x_ref[...] * a_ref[...] + b_ref[...]
    y = jnp.dot(xb, w_ref[...], preferred_element_type=_F32)
    comb_ref[...] = y[:, :GW]
    xud_ref[...] = y[:, HC:GW]
    self_ref[...] = y[:, GW:]


_MM_R = 1000


def _mm_outs():
    return (
        [jax.ShapeDtypeStruct((N, GW), _F32),
         jax.ShapeDtypeStruct((N, XUW), _F32),
         jax.ShapeDtypeStruct((N, C), _F32)],
        [pl.BlockSpec((_MM_R, GW), lambda i: (i, 0)),
         pl.BlockSpec((_MM_R, XUW), lambda i: (i, 0)),
         pl.BlockSpec((_MM_R, C), lambda i: (i, 0))],
    )


def _matmul(x, w):
    out_shape, out_specs = _mm_outs()
    return pl.pallas_call(
        _mm_body,
        grid=(N // _MM_R,),
        in_specs=[pl.BlockSpec((_MM_R, D), lambda i: (i, 0)),
                  pl.BlockSpec((D, KCAT), lambda i: (0, 0))],
        out_specs=out_specs,
        out_shape=out_shape,
    )(x, w)


def _matmul_affine(x, a, b, w):
    out_shape, out_specs = _mm_outs()
    return pl.pallas_call(
        _mm_affine_body,
        grid=(N // _MM_R,),
        in_specs=[pl.BlockSpec((_MM_R, D), lambda i: (i, 0)),
                  pl.BlockSpec((1, D), lambda i: (0, 0)),
                  pl.BlockSpec((1, D), lambda i: (0, 0)),
                  pl.BlockSpec((D, KCAT), lambda i: (0, 0))],
        out_specs=out_specs,
        out_shape=out_shape,
    )(x, a, b, w)


def _post0_body(agg_ref, self_ref, inv_ref, b_ref, hp_ref, stat_ref):
    i = pl.program_id(0)
    s = (agg_ref[0] + agg_ref[1] + self_ref[...]) * inv_ref[...] + b_ref[...]
    hp = jnp.maximum(s, 0.0)
    hp_ref[...] = hp

    @pl.when(i == 0)
    def _init():
        stat_ref[...] = jnp.zeros((8, C), _F32)

    stat_ref[0:1, :] += jnp.sum(hp, axis=0, keepdims=True)
    stat_ref[1:2, :] += jnp.sum(hp * hp, axis=0, keepdims=True)


def _post0(agg, selfm, invb, brow):
    return pl.pallas_call(
        _post0_body,
        grid=(N // _MM_R,),
        in_specs=[pl.BlockSpec((NC, _MM_R, C), lambda i: (0, i, 0)),
                  pl.BlockSpec((_MM_R, C), lambda i: (i, 0)),
                  pl.BlockSpec((_MM_R, C), lambda i: (i, 0)),
                  pl.BlockSpec((1, C), lambda i: (0, 0))],
        out_specs=[pl.BlockSpec((_MM_R, C), lambda i: (i, 0)),
                   pl.BlockSpec((8, C), lambda i: (0, 0))],
        out_shape=[jax.ShapeDtypeStruct((N, C), _F32),
                   jax.ShapeDtypeStruct((8, C), _F32)],
    )(agg, selfm, invb, brow)


def _post1_body(agg_ref, self_ref, inv_ref, b_ref, o_ref):
    o_ref[...] = ((agg_ref[0] + agg_ref[1] + self_ref[...]) * inv_ref[...]
                  + b_ref[...])


def _post1(agg, selfm, invb, brow):
    return pl.pallas_call(
        _post1_body,
        grid=(N // _MM_R,),
        in_specs=[pl.BlockSpec((NC, _MM_R, C), lambda i: (0, i, 0)),
                  pl.BlockSpec((_MM_R, C), lambda i: (i, 0)),
                  pl.BlockSpec((_MM_R, C), lambda i: (i, 0)),
                  pl.BlockSpec((1, C), lambda i: (0, 0))],
        out_specs=pl.BlockSpec((_MM_R, C), lambda i: (i, 0)),
        out_shape=jax.ShapeDtypeStruct((N, C), _F32),
    )(agg, selfm, invb, brow)


def _make_edge_kernel():
    mesh = plsc.VectorSubcoreMesh(core_axis_name="c", subcore_axis_name="s")
    out_type = jax.ShapeDtypeStruct((NC, N, C), _F32)
    scratch = [
        pltpu.VMEM((16,), _F32),
        pltpu.VMEM((EB,), jnp.int32),
        pltpu.VMEM((EB,), jnp.int32),
        pltpu.VMEM((EB, GW), _F32),
        pltpu.VMEM((EB, XUW), _F32),
        pltpu.VMEM((EB, C), _F32),
        pltpu.VMEM_SHARED((NA, C), _F32),
        pltpu.SemaphoreType.DMA,
    ]

    def body(comb, xudt, cpad, srch, dsth, *rest):
        (agg_out, cpad_v, src_v, dst_v, rows_v,
         xud_v, msg_v, agg_sh, sem) = rest

        cid = lax.axis_index("c")
        sid = lax.axis_index("s")
        tid = cid * NS + sid
        base_r = sid * RA
        nblk = jnp.where(sid == NS - 1, RB // 8, RA // 8)

        pltpu.sync_copy(cpad, cpad_v)

        def fill(ref, nrow, ncol, val):
            def row(i, carry):
                for k in range(ncol // 16):
                    ref[i, pl.ds(k * 16, 16)] = jnp.full((16,), val, _F32)
                return carry

            lax.fori_loop(0, nrow, row, 0)

        fill(msg_v, EB, C, 0.0)

        def zinit(j, carry):
            r = base_r + j * 8
            pltpu.sync_copy(msg_v.at[pl.ds(0, 8)], agg_sh.at[pl.ds(r, 8)])
            return carry

        lax.fori_loop(0, nblk, zinit, 0)
        plsc.subcore_barrier()

        cvec = cpad_v[...]

        def chunk(ci, carry):
            base = tid * EPW + ci * EB
            pltpu.sync_copy(srch.at[pl.ds(base, EB)], src_v)
            pltpu.sync_copy(dsth.at[pl.ds(base, EB)], dst_v)
            gat = pltpu.async_copy(comb.at[src_v], rows_v, sem)
            pltpu.sync_copy(xudt.at[dst_v], xud_v)
            gat.wait()

            def msg_one(e, c2):
                lg = xud_v[e, pl.ds(0, 16)] - rows_v[e, pl.ds(HC, 16)] + cvec
                m16 = jnp.broadcast_to(jnp.max(lg), (16,))
                ex = jnp.exp(lg - m16)
                z16 = jnp.broadcast_to(jnp.sum(ex), (16,))
                attn = ex / z16
                lanes = lax.iota(jnp.int32, 16)
                a = [jnp.broadcast_to(
                        jnp.sum(jnp.where(lanes == h, attn, 0.0)), (16,))
                     for h in range(H)]
                for j in range(C // 16):
                    acc = a[0] * rows_v[e, pl.ds(j * 16, 16)]
                    for h in range(1, H):
                        acc = acc + a[h] * rows_v[e, pl.ds(h * C + j * 16, 16)]
                    msg_v[e, pl.ds(j * 16, 16)] = acc
                return c2

            lax.fori_loop(0, EB, msg_one, 0)
            pltpu.sync_copy(msg_v, agg_sh.at[dst_v], add=True)
            return carry

        lax.fori_loop(0, NCHUNK, chunk, 0)

        plsc.subcore_barrier()

        def out_blk(j, carry):
            r = base_r + j * 8
            pltpu.sync_copy(agg_sh.at[pl.ds(r, 8)], msg_v.at[pl.ds(0, 8)])
            pltpu.sync_copy(msg_v.at[pl.ds(0, 8)],
                            agg_out.at[cid, pl.ds(r, 8)])
            return carry

        lax.fori_loop(0, nblk, out_blk, 0)

    return pl.kernel(body, out_type=out_type, mesh=mesh,
                     scratch_types=scratch,
                     compiler_params=pltpu.CompilerParams(
                         needs_layout_passes=False))


_edge_plain = _make_edge_kernel()

ECB = 128
NCHUNK_CNT = 79
EPW_CNT = ECB * NCHUNK_CNT
EPAD_CNT = NW * EPW_CNT


def _make_cnt_kernel():
    mesh = plsc.VectorSubcoreMesh(core_axis_name="c", subcore_axis_name="s")

    @functools.partial(
        pl.kernel, mesh=mesh,
        out_type=jax.ShapeDtypeStruct((NC, N, C), _F32),
        scratch_types=[pltpu.VMEM((ECB,), jnp.int32),
                       pltpu.VMEM((ECB, C), _F32),
                       pltpu.VMEM_SHARED((NA, C), _F32)],
        compiler_params=pltpu.CompilerParams(needs_layout_passes=False))
    def body(dsth, cnt_out, dst_v, ones_v, cnt_sh):
        cid = lax.axis_index("c")
        sid = lax.axis_index("s")
        tid = cid * NS + sid
        base_r = sid * RA
        nblk = jnp.where(sid == NS - 1, RB // 8, RA // 8)

        def fill(val):
            def row(i, c2):
                for k in range(C // 16):
                    ones_v[i, pl.ds(k * 16, 16)] = jnp.full((16,), val, _F32)
                return c2

            lax.fori_loop(0, ECB, row, 0)

        fill(0.0)

        def zinit(j, c2):
            pltpu.sync_copy(ones_v.at[pl.ds(0, 8)],
                            cnt_sh.at[pl.ds(base_r + j * 8, 8)])
            return c2

        lax.fori_loop(0, nblk, zinit, 0)
        fill(1.0)
        plsc.subcore_barrier()

        def chunk(ci, c2):
            base = tid * EPW_CNT + ci * ECB
            pltpu.sync_copy(dsth.at[pl.ds(base, ECB)], dst_v)
            pltpu.sync_copy(ones_v, cnt_sh.at[dst_v], add=True)
            return c2

        lax.fori_loop(0, NCHUNK_CNT, chunk, 0)
        plsc.subcore_barrier()

        def out_blk(j, c2):
            r = base_r + j * 8
            pltpu.sync_copy(cnt_sh.at[pl.ds(r, 8)], ones_v.at[pl.ds(0, 8)])
            pltpu.sync_copy(ones_v.at[pl.ds(0, 8)],
                            cnt_out.at[cid, pl.ds(r, 8)])
            return c2

        lax.fori_loop(0, nblk, out_blk, 0)

    return body


_cnt_kernel = _make_cnt_kernel()


def _prep_weights(W, U, c):
    s = jax.nn.softmax(c)
    w_self = jnp.einsum("dhc,h->dc", W.reshape(-1, H, C), s)
    upad = jnp.pad(U, ((0, 0), (0, XUW - H)))
    wcat = jnp.concatenate([W, upad, w_self], axis=1)
    cpad = jnp.concatenate([c.astype(_F32),
                            jnp.full((16 - H,), -1e30, _F32)])
    return wcat, cpad


def kernel(x, edge_index, W0, U0, c0, b0, gamma0, beta0, W1, U1, c1, b1):
    src = jnp.concatenate([edge_index[0].astype(jnp.int32),
                           jnp.zeros((EPAD - E,), jnp.int32)])
    dst0 = edge_index[1].astype(jnp.int32)
    dst = jnp.concatenate([dst0, jnp.full((EPAD - E,), N, jnp.int32)])
    dstc = jnp.concatenate([dst0, jnp.full((EPAD_CNT - E,), N, jnp.int32)])

    w0cat, cpad0 = _prep_weights(W0, U0, c0)
    w1cat, cpad1 = _prep_weights(W1, U1, c1)

    comb0, xud0, self0 = _matmul(x, w0cat)
    agg0 = _edge_plain(comb0, xud0, cpad0, src, dst)
    cnt = _cnt_kernel(dstc)
    deg = cnt[0, :, 0] + cnt[1, :, 0] + 1.0
    invb = jnp.broadcast_to((1.0 / deg)[:, None], (N, C))
    hp, stat = _post0(agg0, self0, invb, b0[None, :])

    mu = stat[0] / N
    var = stat[1] / N - mu * mu
    aa = gamma0 / jnp.sqrt(var + 1e-5)
    bb = beta0 - mu * aa

    comb1, xud1, self1 = _matmul_affine(hp, aa[None, :], bb[None, :], w1cat)
    agg1 = _edge_plain(comb1, xud1, cpad1, src, dst)
    return _post1(agg1, self1, invb, b1[None, :])


# speedup vs baseline: 5.0252x; 1.3600x over previous
import functools

import jax
import jax.numpy as jnp
from jax import lax
from jax.experimental import pallas as pl
from jax.experimental.pallas import tpu as pltpu
from jax.experimental.pallas import tpu_sc as plsc

N = 10000
E = 320000
D = 128
H = 4
C = 128
HC = H * C
XUW = 128
GW = HC + XUW
KCAT = GW + C

NC, NS = 2, 16
NW = NC * NS
EB = 24
NCHUNK = 418
EPW = EB * NCHUNK
EPAD = NW * EPW
NA = N + 8
RA = 632
RB = N - (NS - 1) * RA

_F32 = jnp.float32


def _mm_body(x_ref, w_ref, comb_ref, xud_ref, self_ref):
    y = jnp.dot(x_ref[...], w_ref[...], preferred_element_type=_F32)
    comb_ref[...] = y[:, :GW]
    xud_ref[...] = y[:, HC:GW]
    self_ref[...] = y[:, GW:]


def _mm_affine_body(x_ref, a_ref, b_ref, w_ref, comb_ref, xud_ref, self_ref):
    xb = x_ref[...] * a_ref[...] + b_ref[...]
    y = jnp.dot(xb, w_ref[...], preferred_element_type=_F32)
    comb_ref[...] = y[:, :GW]
    xud_ref[...] = y[:, HC:GW]
    self_ref[...] = y[:, GW:]


_MM_R = 1000


def _mm_outs():
    return (
        [jax.ShapeDtypeStruct((N, GW), _F32),
         jax.ShapeDtypeStruct((N, XUW), _F32),
         jax.ShapeDtypeStruct((N, C), _F32)],
        [pl.BlockSpec((_MM_R, GW), lambda i: (i, 0)),
         pl.BlockSpec((_MM_R, XUW), lambda i: (i, 0)),
         pl.BlockSpec((_MM_R, C), lambda i: (i, 0))],
    )


def _matmul(x, w):
    out_shape, out_specs = _mm_outs()
    return pl.pallas_call(
        _mm_body,
        grid=(N // _MM_R,),
        in_specs=[pl.BlockSpec((_MM_R, D), lambda i: (i, 0)),
                  pl.BlockSpec((D, KCAT), lambda i: (0, 0))],
        out_specs=out_specs,
        out_shape=out_shape,
    )(x, w)


def _matmul_affine(x, a, b, w):
    out_shape, out_specs = _mm_outs()
    return pl.pallas_call(
        _mm_affine_body,
        grid=(N // _MM_R,),
        in_specs=[pl.BlockSpec((_MM_R, D), lambda i: (i, 0)),
                  pl.BlockSpec((1, D), lambda i: (0, 0)),
                  pl.BlockSpec((1, D), lambda i: (0, 0)),
                  pl.BlockSpec((D, KCAT), lambda i: (0, 0))],
        out_specs=out_specs,
        out_shape=out_shape,
    )(x, a, b, w)


def _post0_body(agg_ref, self_ref, inv_ref, b_ref, hp_ref, stat_ref):
    i = pl.program_id(0)
    s = (agg_ref[0] + agg_ref[1] + self_ref[...]) * inv_ref[...] + b_ref[...]
    hp = jnp.maximum(s, 0.0)
    hp_ref[...] = hp

    @pl.when(i == 0)
    def _init():
        stat_ref[...] = jnp.zeros((8, C), _F32)

    stat_ref[0:1, :] += jnp.sum(hp, axis=0, keepdims=True)
    stat_ref[1:2, :] += jnp.sum(hp * hp, axis=0, keepdims=True)


def _post0(agg, selfm, invb, brow):
    return pl.pallas_call(
        _post0_body,
        grid=(N // _MM_R,),
        in_specs=[pl.BlockSpec((NC, _MM_R, C), lambda i: (0, i, 0)),
                  pl.BlockSpec((_MM_R, C), lambda i: (i, 0)),
                  pl.BlockSpec((_MM_R, C), lambda i: (i, 0)),
                  pl.BlockSpec((1, C), lambda i: (0, 0))],
        out_specs=[pl.BlockSpec((_MM_R, C), lambda i: (i, 0)),
                   pl.BlockSpec((8, C), lambda i: (0, 0))],
        out_shape=[jax.ShapeDtypeStruct((N, C), _F32),
                   jax.ShapeDtypeStruct((8, C), _F32)],
    )(agg, selfm, invb, brow)


def _post1_body(agg_ref, self_ref, inv_ref, b_ref, o_ref):
    o_ref[...] = ((agg_ref[0] + agg_ref[1] + self_ref[...]) * inv_ref[...]
                  + b_ref[...])


def _post1(agg, selfm, invb, brow):
    return pl.pallas_call(
        _post1_body,
        grid=(N // _MM_R,),
        in_specs=[pl.BlockSpec((NC, _MM_R, C), lambda i: (0, i, 0)),
                  pl.BlockSpec((_MM_R, C), lambda i: (i, 0)),
                  pl.BlockSpec((_MM_R, C), lambda i: (i, 0)),
                  pl.BlockSpec((1, C), lambda i: (0, 0))],
        out_specs=pl.BlockSpec((_MM_R, C), lambda i: (i, 0)),
        out_shape=jax.ShapeDtypeStruct((N, C), _F32),
    )(agg, selfm, invb, brow)


def _make_edge_kernel():
    mesh = plsc.VectorSubcoreMesh(core_axis_name="c", subcore_axis_name="s")
    out_type = jax.ShapeDtypeStruct((NC, N, C), _F32)
    scratch = [
        pltpu.VMEM((16,), _F32),
        pltpu.VMEM((EB,), jnp.int32),
        pltpu.VMEM((EB,), jnp.int32),
        pltpu.VMEM((EB,), jnp.int32),
        pltpu.VMEM((EB,), jnp.int32),
        pltpu.VMEM((EB, GW), _F32),
        pltpu.VMEM((EB, GW), _F32),
        pltpu.VMEM((EB, XUW), _F32),
        pltpu.VMEM((EB, XUW), _F32),
        pltpu.VMEM((EB, C), _F32),
        pltpu.VMEM_SHARED((NA, C), _F32),
        pltpu.SemaphoreType.DMA,
        pltpu.SemaphoreType.DMA,
        pltpu.SemaphoreType.DMA,
        pltpu.SemaphoreType.DMA,
    ]

    def body(comb, xudt, cpad, srch, dsth, *rest):
        (agg_out, cpad_v, src0, src1, dst0, dst1, rows0, rows1,
         xud0, xud1, msg_v, agg_sh, semi0, semi1, semg0, semg1) = rest
        SRC = (src0, src1)
        DST = (dst0, dst1)
        ROWS = (rows0, rows1)
        XUD = (xud0, xud1)
        SEMI = (semi0, semi1)
        SEMG = (semg0, semg1)

        cid = lax.axis_index("c")
        sid = lax.axis_index("s")
        tid = cid * NS + sid
        base_r = sid * RA
        nblk = jnp.where(sid == NS - 1, RB // 8, RA // 8)

        pltpu.sync_copy(cpad, cpad_v)

        def fill(ref, nrow, ncol, val):
            def row(i, carry):
                for k in range(ncol // 16):
                    ref[i, pl.ds(k * 16, 16)] = jnp.full((16,), val, _F32)
                return carry

            lax.fori_loop(0, nrow, row, 0)

        fill(msg_v, EB, C, 0.0)

        def zinit(j, carry):
            r = base_r + j * 8
            pltpu.sync_copy(msg_v.at[pl.ds(0, 8)], agg_sh.at[pl.ds(r, 8)])
            return carry

        lax.fori_loop(0, nblk, zinit, 0)
        plsc.subcore_barrier()

        cvec = cpad_v[...]

        def idx_fire(ci, s):
            base = tid * EPW + ci * EB
            pltpu.async_copy(srch.at[pl.ds(base, EB)], SRC[s], SEMI[s])
            pltpu.async_copy(dsth.at[pl.ds(base, EB)], DST[s], SEMI[s])

        def idx_wait(ci, s):
            base = tid * EPW + ci * EB
            pltpu.make_async_copy(srch.at[pl.ds(base, EB)], SRC[s],
                                  SEMI[s]).wait()
            pltpu.make_async_copy(dsth.at[pl.ds(base, EB)], DST[s],
                                  SEMI[s]).wait()

        def gat_fire(s):
            pltpu.async_copy(comb.at[SRC[s]], ROWS[s], SEMG[s])
            pltpu.async_copy(xudt.at[DST[s]], XUD[s], SEMG[s])

        def gat_wait(s):
            pltpu.make_async_copy(comb.at[SRC[s]], ROWS[s], SEMG[s]).wait()
            pltpu.make_async_copy(xudt.at[DST[s]], XUD[s], SEMG[s]).wait()

        def compute(s):
            rows_v = ROWS[s]
            xud_v = XUD[s]

            def msg_one(e, c2):
                lg = xud_v[e, pl.ds(0, 16)] - rows_v[e, pl.ds(HC, 16)] + cvec
                m16 = jnp.broadcast_to(jnp.max(lg), (16,))
                ex = jnp.exp(lg - m16)
                z16 = jnp.broadcast_to(jnp.sum(ex), (16,))
                attn = ex / z16
                lanes = lax.iota(jnp.int32, 16)
                a = [jnp.broadcast_to(
                        jnp.sum(jnp.where(lanes == h, attn, 0.0)), (16,))
                     for h in range(H)]
                for j in range(C // 16):
                    acc = a[0] * rows_v[e, pl.ds(j * 16, 16)]
                    for h in range(1, H):
                        acc = acc + a[h] * rows_v[e, pl.ds(h * C + j * 16, 16)]
                    msg_v[e, pl.ds(j * 16, 16)] = acc
                return c2

            lax.fori_loop(0, EB, msg_one, 0)
            pltpu.sync_copy(msg_v, agg_sh.at[DST[s]], add=True)

        idx_fire(0, 0)
        idx_wait(0, 0)
        gat_fire(0)
        idx_fire(1, 1)

        def super_body(sblk, carry):
            for p in range(2):
                ci = sblk * 2 + p
                q = 1 - p

                @pl.when(ci + 1 < NCHUNK)
                def _prefetch():
                    idx_wait(ci + 1, q)
                    gat_fire(q)

                gat_wait(p)
                compute(p)

                @pl.when(ci + 2 < NCHUNK)
                def _nextidx():
                    idx_fire(ci + 2, p)
            return carry

        lax.fori_loop(0, NCHUNK // 2, super_body, 0)

        plsc.subcore_barrier()

        def out_blk(j, carry):
            r = base_r + j * 8
            pltpu.sync_copy(agg_sh.at[pl.ds(r, 8)], msg_v.at[pl.ds(0, 8)])
            pltpu.sync_copy(msg_v.at[pl.ds(0, 8)],
                            agg_out.at[cid, pl.ds(r, 8)])
            return carry

        lax.fori_loop(0, nblk, out_blk, 0)

    return pl.kernel(body, out_type=out_type, mesh=mesh,
                     scratch_types=scratch,
                     compiler_params=pltpu.CompilerParams(
                         needs_layout_passes=False))


_edge_plain = _make_edge_kernel()

ECB = 128
NCHUNK_CNT = 79
EPW_CNT = ECB * NCHUNK_CNT
EPAD_CNT = NW * EPW_CNT


def _make_cnt_kernel():
    mesh = plsc.VectorSubcoreMesh(core_axis_name="c", subcore_axis_name="s")

    @functools.partial(
        pl.kernel, mesh=mesh,
        out_type=jax.ShapeDtypeStruct((NC, N, C), _F32),
        scratch_types=[pltpu.VMEM((ECB,), jnp.int32),
                       pltpu.VMEM((ECB, C), _F32),
                       pltpu.VMEM_SHARED((NA, C), _F32)],
        compiler_params=pltpu.CompilerParams(needs_layout_passes=False))
    def body(dsth, cnt_out, dst_v, ones_v, cnt_sh):
        cid = lax.axis_index("c")
        sid = lax.axis_index("s")
        tid = cid * NS + sid
        base_r = sid * RA
        nblk = jnp.where(sid == NS - 1, RB // 8, RA // 8)

        def fill(val):
            def row(i, c2):
                for k in range(C // 16):
                    ones_v[i, pl.ds(k * 16, 16)] = jnp.full((16,), val, _F32)
                return c2

            lax.fori_loop(0, ECB, row, 0)

        fill(0.0)

        def zinit(j, c2):
            pltpu.sync_copy(ones_v.at[pl.ds(0, 8)],
                            cnt_sh.at[pl.ds(base_r + j * 8, 8)])
            return c2

        lax.fori_loop(0, nblk, zinit, 0)
        fill(1.0)
        plsc.subcore_barrier()

        def chunk(ci, c2):
            base = tid * EPW_CNT + ci * ECB
            pltpu.sync_copy(dsth.at[pl.ds(base, ECB)], dst_v)
            pltpu.sync_copy(ones_v, cnt_sh.at[dst_v], add=True)
            return c2

        lax.fori_loop(0, NCHUNK_CNT, chunk, 0)
        plsc.subcore_barrier()

        def out_blk(j, c2):
            r = base_r + j * 8
            pltpu.sync_copy(cnt_sh.at[pl.ds(r, 8)], ones_v.at[pl.ds(0, 8)])
            pltpu.sync_copy(ones_v.at[pl.ds(0, 8)],
                            cnt_out.at[cid, pl.ds(r, 8)])
            return c2

        lax.fori_loop(0, nblk, out_blk, 0)

    return body


_cnt_kernel = _make_cnt_kernel()


def _prep_weights(W, U, c):
    s = jax.nn.softmax(c)
    w_self = jnp.einsum("dhc,h->dc", W.reshape(-1, H, C), s)
    upad = jnp.pad(U, ((0, 0), (0, XUW - H)))
    wcat = jnp.concatenate([W, upad, w_self], axis=1)
    cpad = jnp.concatenate([c.astype(_F32),
                            jnp.full((16 - H,), -1e30, _F32)])
    return wcat, cpad


def kernel(x, edge_index, W0, U0, c0, b0, gamma0, beta0, W1, U1, c1, b1):
    src = jnp.concatenate([edge_index[0].astype(jnp.int32),
                           jnp.zeros((EPAD - E,), jnp.int32)])
    dst0 = edge_index[1].astype(jnp.int32)
    dst = jnp.concatenate([dst0, jnp.full((EPAD - E,), N, jnp.int32)])
    dstc = jnp.concatenate([dst0, jnp.full((EPAD_CNT - E,), N, jnp.int32)])

    w0cat, cpad0 = _prep_weights(W0, U0, c0)
    w1cat, cpad1 = _prep_weights(W1, U1, c1)

    comb0, xud0, self0 = _matmul(x, w0cat)
    agg0 = _edge_plain(comb0, xud0, cpad0, src, dst)
    cnt = _cnt_kernel(dstc)
    deg = cnt[0, :, 0] + cnt[1, :, 0] + 1.0
    invb = jnp.broadcast_to((1.0 / deg)[:, None], (N, C))
    hp, stat = _post0(agg0, self0, invb, b0[None, :])

    mu = stat[0] / N
    var = stat[1] / N - mu * mu
    aa = gamma0 / jnp.sqrt(var + 1e-5)
    bb = beta0 - mu * aa

    comb1, xud1, self1 = _matmul_affine(hp, aa[None, :], bb[None, :], w1cat)
    agg1 = _edge_plain(comb1, xud1, cpad1, src, dst)
    return _post1(agg1, self1, invb, b1[None, :])


# speedup vs baseline: 5.9840x; 1.1908x over previous
import functools

import jax
import jax.numpy as jnp
from jax import lax
from jax.experimental import pallas as pl
from jax.experimental.pallas import tpu as pltpu
from jax.experimental.pallas import tpu_sc as plsc

N = 10000
E = 320000
D = 128
H = 4
C = 128
HC = H * C
XUW = 128
GW = HC + XUW
KCAT = GW + C

NC, NS = 2, 16
NW = NC * NS
EB = 24
NCHUNK = 420
EPW = EB * NCHUNK
EPAD = NW * EPW
NA = N + 8
RA = 632
RB = N - (NS - 1) * RA

_F32 = jnp.float32


def _mm_body(x_ref, w_ref, comb_ref, xud_ref, self_ref):
    y = jnp.dot(x_ref[...], w_ref[...], preferred_element_type=_F32)
    comb_ref[...] = y[:, :GW]
    xud_ref[...] = y[:, HC:GW]
    self_ref[...] = y[:, GW:]


def _mm_affine_body(x_ref, a_ref, b_ref, w_ref, comb_ref, xud_ref, self_ref):
    xb = x_ref[...] * a_ref[...] + b_ref[...]
    y = jnp.dot(xb, w_ref[...], preferred_element_type=_F32)
    comb_ref[...] = y[:, :GW]
    xud_ref[...] = y[:, HC:GW]
    self_ref[...] = y[:, GW:]


_MM_R = 1000


def _mm_outs():
    return (
        [jax.ShapeDtypeStruct((N, GW), _F32),
         jax.ShapeDtypeStruct((N, XUW), _F32),
         jax.ShapeDtypeStruct((N, C), _F32)],
        [pl.BlockSpec((_MM_R, GW), lambda i: (i, 0)),
         pl.BlockSpec((_MM_R, XUW), lambda i: (i, 0)),
         pl.BlockSpec((_MM_R, C), lambda i: (i, 0))],
    )


def _matmul(x, w):
    out_shape, out_specs = _mm_outs()
    return pl.pallas_call(
        _mm_body,
        grid=(N // _MM_R,),
        in_specs=[pl.BlockSpec((_MM_R, D), lambda i: (i, 0)),
                  pl.BlockSpec((D, KCAT), lambda i: (0, 0))],
        out_specs=out_specs,
        out_shape=out_shape,
    )(x, w)


def _matmul_affine(x, a, b, w):
    out_shape, out_specs = _mm_outs()
    return pl.pallas_call(
        _mm_affine_body,
        grid=(N // _MM_R,),
        in_specs=[pl.BlockSpec((_MM_R, D), lambda i: (i, 0)),
                  pl.BlockSpec((1, D), lambda i: (0, 0)),
                  pl.BlockSpec((1, D), lambda i: (0, 0)),
                  pl.BlockSpec((D, KCAT), lambda i: (0, 0))],
        out_specs=out_specs,
        out_shape=out_shape,
    )(x, a, b, w)


def _post0_body(agg_ref, self_ref, inv_ref, b_ref, hp_ref, stat_ref):
    i = pl.program_id(0)
    s = (agg_ref[0] + agg_ref[1] + self_ref[...]) * inv_ref[...] + b_ref[...]
    hp = jnp.maximum(s, 0.0)
    hp_ref[...] = hp

    @pl.when(i == 0)
    def _init():
        stat_ref[...] = jnp.zeros((8, C), _F32)

    stat_ref[0:1, :] += jnp.sum(hp, axis=0, keepdims=True)
    stat_ref[1:2, :] += jnp.sum(hp * hp, axis=0, keepdims=True)


def _post0(agg, selfm, invb, brow):
    return pl.pallas_call(
        _post0_body,
        grid=(N // _MM_R,),
        in_specs=[pl.BlockSpec((NC, _MM_R, C), lambda i: (0, i, 0)),
                  pl.BlockSpec((_MM_R, C), lambda i: (i, 0)),
                  pl.BlockSpec((_MM_R, C), lambda i: (i, 0)),
                  pl.BlockSpec((1, C), lambda i: (0, 0))],
        out_specs=[pl.BlockSpec((_MM_R, C), lambda i: (i, 0)),
                   pl.BlockSpec((8, C), lambda i: (0, 0))],
        out_shape=[jax.ShapeDtypeStruct((N, C), _F32),
                   jax.ShapeDtypeStruct((8, C), _F32)],
    )(agg, selfm, invb, brow)


def _post1_body(agg_ref, self_ref, inv_ref, b_ref, o_ref):
    o_ref[...] = ((agg_ref[0] + agg_ref[1] + self_ref[...]) * inv_ref[...]
                  + b_ref[...])


def _post1(agg, selfm, invb, brow):
    return pl.pallas_call(
        _post1_body,
        grid=(N // _MM_R,),
        in_specs=[pl.BlockSpec((NC, _MM_R, C), lambda i: (0, i, 0)),
                  pl.BlockSpec((_MM_R, C), lambda i: (i, 0)),
                  pl.BlockSpec((_MM_R, C), lambda i: (i, 0)),
                  pl.BlockSpec((1, C), lambda i: (0, 0))],
        out_specs=pl.BlockSpec((_MM_R, C), lambda i: (i, 0)),
        out_shape=jax.ShapeDtypeStruct((N, C), _F32),
    )(agg, selfm, invb, brow)


def _make_edge_kernel():
    mesh = plsc.VectorSubcoreMesh(core_axis_name="c", subcore_axis_name="s")
    out_type = jax.ShapeDtypeStruct((NC, N, C), _F32)
    scratch = (
        [pltpu.VMEM((16,), _F32),
         pltpu.VMEM((16,), _F32)] +
        [pltpu.VMEM((EB,), jnp.int32)] * 2 +
        [pltpu.VMEM((EB,), jnp.int32)] * 4 +
        [pltpu.VMEM((EB, GW), _F32)] * 2 +
        [pltpu.VMEM((EB, XUW), _F32)] * 2 +
        [pltpu.VMEM((EB, C), _F32)] * 2 +
        [pltpu.VMEM_SHARED((NA, C), _F32)] +
        [pltpu.SemaphoreType.DMA] * 6
    )

    def body(comb, xudt, cpad, srch, dsth, *rest):
        (agg_out, cpad_v, att_v, src0, src1, dst0, dst1, dst2, dst3,
         rows0, rows1, xud0, xud1, msg0, msg1, agg_sh,
         semi0, semi1, semg0, semg1, sems0, sems1) = rest
        SRC = (src0, src1)
        DST = (dst0, dst1, dst2, dst3)
        ROWS = (rows0, rows1)
        XUD = (xud0, xud1)
        MSG = (msg0, msg1)
        SEMI = (semi0, semi1)
        SEMG = (semg0, semg1)
        SEMS = (sems0, sems1)

        cid = lax.axis_index("c")
        sid = lax.axis_index("s")
        tid = cid * NS + sid
        base_r = sid * RA
        nblk = jnp.where(sid == NS - 1, RB // 8, RA // 8)

        pltpu.sync_copy(cpad, cpad_v)

        def fill(ref, nrow, ncol, val):
            def row(i, carry):
                for k in range(ncol // 16):
                    ref[i, pl.ds(k * 16, 16)] = jnp.full((16,), val, _F32)
                return carry

            lax.fori_loop(0, nrow, row, 0)

        fill(msg0, EB, C, 0.0)

        def zinit(j, carry):
            r = base_r + j * 8
            pltpu.sync_copy(msg0.at[pl.ds(0, 8)], agg_sh.at[pl.ds(r, 8)])
            return carry

        lax.fori_loop(0, nblk, zinit, 0)
        plsc.subcore_barrier()

        cvec = cpad_v[...]

        def idx_fire(ci, p, d):
            base = tid * EPW + ci * EB
            pltpu.async_copy(srch.at[pl.ds(base, EB)], SRC[p], SEMI[p])
            pltpu.async_copy(dsth.at[pl.ds(base, EB)], DST[d], SEMI[p])

        def idx_wait(ci, p, d):
            base = tid * EPW + ci * EB
            pltpu.make_async_copy(srch.at[pl.ds(base, EB)], SRC[p],
                                  SEMI[p]).wait()
            pltpu.make_async_copy(dsth.at[pl.ds(base, EB)], DST[d],
                                  SEMI[p]).wait()

        def gat_fire(p, d):
            pltpu.async_copy(comb.at[SRC[p]], ROWS[p], SEMG[p])
            pltpu.async_copy(xudt.at[DST[d]], XUD[p], SEMG[p])

        def gat_wait(p, d):
            pltpu.make_async_copy(comb.at[SRC[p]], ROWS[p], SEMG[p]).wait()
            pltpu.make_async_copy(xudt.at[DST[d]], XUD[p], SEMG[p]).wait()

        def scat_wait(p, d):
            pltpu.make_async_copy(MSG[p], agg_sh.at[DST[d]], SEMS[p]).wait()

        def compute(p):
            rows_v = ROWS[p]
            xud_v = XUD[p]
            msg_v = MSG[p]

            def msg_one(e, c2):
                lg = xud_v[e, pl.ds(0, 16)] - rows_v[e, pl.ds(HC, 16)] + cvec
                ex = jnp.exp(lg)
                zi = 1.0 / jnp.broadcast_to(jnp.sum(ex), (16,))
                lanes = lax.iota(jnp.int32, 16)
                a = [jnp.broadcast_to(
                        jnp.sum(jnp.where(lanes == h, ex, 0.0)), (16,)) * zi
                     for h in range(H)]
                for j in range(C // 16):
                    acc = a[0] * rows_v[e, pl.ds(j * 16, 16)]
                    for h in range(1, H):
                        acc = acc + a[h] * rows_v[e, pl.ds(h * C + j * 16, 16)]
                    msg_v[e, pl.ds(j * 16, 16)] = acc
                return c2

            lax.fori_loop(0, EB, msg_one, 0)

        idx_fire(0, 0, 0)
        idx_wait(0, 0, 0)
        gat_fire(0, 0)
        idx_fire(1, 1, 1)

        def super_body(sblk, carry):
            for u in range(4):
                ci = sblk * 4 + u
                p = u % 2
                q = 1 - p
                d = u
                dn = (u + 1) % 4
                dp = (u + 2) % 4

                @pl.when(ci + 1 < NCHUNK)
                def _prefetch():
                    idx_wait(ci + 1, q, dn)
                    gat_fire(q, dn)

                gat_wait(p, d)

                @pl.when(ci >= 2)
                def _drain():
                    scat_wait(p, dp)

                compute(p)
                pltpu.async_copy(MSG[p], agg_sh.at[DST[d]], SEMS[p],
                                 add=True)

                @pl.when(ci + 2 < NCHUNK)
                def _nextidx():
                    idx_fire(ci + 2, p, dp)
            return carry

        lax.fori_loop(0, NCHUNK // 4, super_body, 0)

        scat_wait(0, 2)
        scat_wait(1, 3)

        plsc.subcore_barrier()

        def out_blk(j, carry):
            r = base_r + j * 8
            pltpu.sync_copy(agg_sh.at[pl.ds(r, 8)], msg0.at[pl.ds(0, 8)])
            pltpu.sync_copy(msg0.at[pl.ds(0, 8)],
                            agg_out.at[cid, pl.ds(r, 8)])
            return carry

        lax.fori_loop(0, nblk, out_blk, 0)

    return pl.kernel(body, out_type=out_type, mesh=mesh,
                     scratch_types=scratch,
                     compiler_params=pltpu.CompilerParams(
                         needs_layout_passes=False))


_edge_plain = _make_edge_kernel()

ECB = 128
NCHUNK_CNT = 79
EPW_CNT = ECB * NCHUNK_CNT
EPAD_CNT = NW * EPW_CNT


def _make_cnt_kernel():
    mesh = plsc.VectorSubcoreMesh(core_axis_name="c", subcore_axis_name="s")

    @functools.partial(
        pl.kernel, mesh=mesh,
        out_type=jax.ShapeDtypeStruct((NC, N, C), _F32),
        scratch_types=[pltpu.VMEM((ECB,), jnp.int32),
                       pltpu.VMEM((ECB, C), _F32),
                       pltpu.VMEM_SHARED((NA, C), _F32)],
        compiler_params=pltpu.CompilerParams(needs_layout_passes=False))
    def body(dsth, cnt_out, dst_v, ones_v, cnt_sh):
        cid = lax.axis_index("c")
        sid = lax.axis_index("s")
        tid = cid * NS + sid
        base_r = sid * RA
        nblk = jnp.where(sid == NS - 1, RB // 8, RA // 8)

        def fill(val):
            def row(i, c2):
                for k in range(C // 16):
                    ones_v[i, pl.ds(k * 16, 16)] = jnp.full((16,), val, _F32)
                return c2

            lax.fori_loop(0, ECB, row, 0)

        fill(0.0)

        def zinit(j, c2):
            pltpu.sync_copy(ones_v.at[pl.ds(0, 8)],
                            cnt_sh.at[pl.ds(base_r + j * 8, 8)])
            return c2

        lax.fori_loop(0, nblk, zinit, 0)
        fill(1.0)
        plsc.subcore_barrier()

        def chunk(ci, c2):
            base = tid * EPW_CNT + ci * ECB
            pltpu.sync_copy(dsth.at[pl.ds(base, ECB)], dst_v)
            pltpu.sync_copy(ones_v, cnt_sh.at[dst_v], add=True)
            return c2

        lax.fori_loop(0, NCHUNK_CNT, chunk, 0)
        plsc.subcore_barrier()

        def out_blk(j, c2):
            r = base_r + j * 8
            pltpu.sync_copy(cnt_sh.at[pl.ds(r, 8)], ones_v.at[pl.ds(0, 8)])
            pltpu.sync_copy(ones_v.at[pl.ds(0, 8)],
                            cnt_out.at[cid, pl.ds(r, 8)])
            return c2

        lax.fori_loop(0, nblk, out_blk, 0)

    return body


_cnt_kernel = _make_cnt_kernel()


def _prep_weights(W, U, c):
    s = jax.nn.softmax(c)
    w_self = jnp.einsum("dhc,h->dc", W.reshape(-1, H, C), s)
    upad = jnp.pad(U, ((0, 0), (0, XUW - H)))
    wcat = jnp.concatenate([W, upad, w_self], axis=1)
    cpad = jnp.concatenate([c.astype(_F32),
                            jnp.full((16 - H,), -1e30, _F32)])
    return wcat, cpad


def kernel(x, edge_index, W0, U0, c0, b0, gamma0, beta0, W1, U1, c1, b1):
    src = jnp.concatenate([edge_index[0].astype(jnp.int32),
                           jnp.zeros((EPAD - E,), jnp.int32)])
    dst0 = edge_index[1].astype(jnp.int32)
    dst = jnp.concatenate([dst0, jnp.full((EPAD - E,), N, jnp.int32)])
    dstc = jnp.concatenate([dst0, jnp.full((EPAD_CNT - E,), N, jnp.int32)])

    w0cat, cpad0 = _prep_weights(W0, U0, c0)
    w1cat, cpad1 = _prep_weights(W1, U1, c1)

    comb0, xud0, self0 = _matmul(x, w0cat)
    agg0 = _edge_plain(comb0, xud0, cpad0, src, dst)
    cnt = _cnt_kernel(dstc)
    deg = cnt[0, :, 0] + cnt[1, :, 0] + 1.0
    invb = jnp.broadcast_to((1.0 / deg)[:, None], (N, C))
    hp, stat = _post0(agg0, self0, invb, b0[None, :])

    mu = stat[0] / N
    var = stat[1] / N - mu * mu
    aa = gamma0 / jnp.sqrt(var + 1e-5)
    bb = beta0 - mu * aa

    comb1, xud1, self1 = _matmul_affine(hp, aa[None, :], bb[None, :], w1cat)
    agg1 = _edge_plain(comb1, xud1, cpad1, src, dst)
    return _post1(agg1, self1, invb, b1[None, :])
